# baseline (speedup 1.0000x reference)
import functools

import jax
import jax.numpy as jnp
from jax import lax
from jax.experimental import pallas as pl
from jax.experimental.pallas import tpu as pltpu
from jax.experimental.pallas import tpu_sc as plsc

_NC = 2
_NS = 16
_NW = _NC * _NS

_D = 128
_WIDE = 144
_IB = 128
_FIRE = 4
_FIRE_S = 2


def _silu(x):
    return x / (1.0 + jnp.exp(-x))


def _k1_body(nf_ref, coord_ref, wa_ref, wb_ref, be1_ref, ta_ref, tb_ref):
    nf = nf_ref[...]
    cd = coord_ref[...]
    bn = nf.shape[0]
    zpad = jnp.zeros((bn, _WIDE - _D - 3), jnp.float32)
    a = jnp.dot(nf, wa_ref[...], preferred_element_type=jnp.float32) + be1_ref[...]
    b = jnp.dot(nf, wb_ref[...], preferred_element_type=jnp.float32)
    ta_ref[...] = jnp.concatenate([a, cd, zpad], axis=1)
    tb_ref[...] = jnp.concatenate([b, -cd, zpad], axis=1)


def _make_tables(node_feat, coord, W_e1a, W_e1b, b_e1, bn=400):
    n = node_feat.shape[0]
    grid = n // bn
    return pl.pallas_call(
        _k1_body,
        grid=(grid,),
        in_specs=[
            pl.BlockSpec((bn, _D), lambda i: (i, 0)),
            pl.BlockSpec((bn, 3), lambda i: (i, 0)),
            pl.BlockSpec((_D, _D), lambda i: (0, 0)),
            pl.BlockSpec((_D, _D), lambda i: (0, 0)),
            pl.BlockSpec((1, _D), lambda i: (0, 0)),
        ],
        out_specs=[
            pl.BlockSpec((bn, _WIDE), lambda i: (i, 0)),
            pl.BlockSpec((bn, _WIDE), lambda i: (i, 0)),
        ],
        out_shape=[
            jax.ShapeDtypeStruct((n, _WIDE), jnp.float32),
            jax.ShapeDtypeStruct((n, _WIDE), jnp.float32),
        ],
    )(node_feat, coord, W_e1a, W_e1b, b_e1)


def _gather_body(bpw, ta_ref, tb_ref, row_ref, col_ref, out_ref,
                 rbuf, cbuf, sbuf, gsem):
    w = lax.axis_index("s") * _NC + lax.axis_index("c")
    base = w * bpw
    pltpu.sync_copy(row_ref.at[pl.ds(base, bpw)], rbuf)
    pltpu.sync_copy(col_ref.at[pl.ds(base, bpw)], cbuf)

    def chunk(q, carry):
        blk0 = q * _FIRE
        descs = [
            pltpu.async_copy(ta_ref.at[rbuf.at[blk0 + j]],
                             sbuf.at[pl.ds(j * _IB, _IB)], gsem)
            for j in range(_FIRE)
        ]
        for d in descs:
            d.wait()
        descs = [
            pltpu.async_copy(tb_ref.at[cbuf.at[blk0 + j]],
                             sbuf.at[pl.ds(j * _IB, _IB)], gsem, add=True)
            for j in range(_FIRE)
        ]
        for d in descs:
            d.wait()
        pltpu.sync_copy(sbuf,
                        out_ref.at[pl.ds((base + blk0) * _IB, _FIRE * _IB)])
        return carry

    lax.fori_loop(0, bpw // _FIRE, chunk, 0)


def _sc_gather(ta, tb, row2d, col2d):
    nblk = row2d.shape[0]
    bpw = nblk // _NW
    e_pad = nblk * _IB
    wid = ta.shape[1]
    mesh = plsc.VectorSubcoreMesh(core_axis_name="c", subcore_axis_name="s")
    return pl.kernel(
        functools.partial(_gather_body, bpw),
        out_type=jax.ShapeDtypeStruct((e_pad, wid), ta.dtype),
        mesh=mesh,
        compiler_params=pltpu.CompilerParams(use_tc_tiling_on_sc=False),
        scratch_types=[
            pltpu.VMEM((bpw, _IB), jnp.int32),
            pltpu.VMEM((bpw, _IB), jnp.int32),
            pltpu.VMEM((_FIRE * _IB, wid), ta.dtype),
            pltpu.SemaphoreType.DMA,
        ],
    )(ta, tb, row2d, col2d)


def _k2_body(sp_ref, wr_ref, we2_ref, be2_ref, wc1_ref, bc1_ref, wc2_ref,
             out_ref):
    sp = sp_ref[...].astype(jnp.float32)
    bn = sp.shape[0]
    x = sp[:, :_D]
    cd = sp[:, _D:_D + 3]
    r = jnp.sum(cd * cd, axis=1, keepdims=True)
    pre = x + r * wr_ref[...]
    h1 = _silu(pre)
    m = _silu(jnp.dot(h1, we2_ref[...], preferred_element_type=jnp.float32)
              + be2_ref[...])
    g1 = _silu(jnp.dot(m, wc1_ref[...], preferred_element_type=jnp.float32)
               + bc1_ref[...])
    gate = jnp.dot(g1, wc2_ref[...], preferred_element_type=jnp.float32)
    trans = cd * gate
    ones = jnp.ones((bn, 1), jnp.float32)
    zpad = jnp.zeros((bn, _WIDE - _D - 4), jnp.float32)
    out_ref[...] = jnp.concatenate([m, trans, ones, zpad], axis=1)


def _edge_mlp(sp, w_r, W_e2, b_e2, W_c1, b_c1, W_c2, bn=2048):
    e_pad = sp.shape[0]
    grid = e_pad // bn
    wid = sp.shape[1]
    return pl.pallas_call(
        _k2_body,
        grid=(grid,),
        in_specs=[
            pl.BlockSpec((bn, wid), lambda i: (i, 0)),
            pl.BlockSpec((1, _D), lambda i: (0, 0)),
            pl.BlockSpec((_D, _D), lambda i: (0, 0)),
            pl.BlockSpec((1, _D), lambda i: (0, 0)),
            pl.BlockSpec((_D, _D), lambda i: (0, 0)),
            pl.BlockSpec((1, _D), lambda i: (0, 0)),
            pl.BlockSpec((_D, 1), lambda i: (0, 0)),
        ],
        out_specs=pl.BlockSpec((bn, _WIDE), lambda i: (i, 0)),
        out_shape=jax.ShapeDtypeStruct((e_pad, _WIDE), jnp.float32),
    )(sp, w_r, W_e2, b_e2, W_c1, b_c1, W_c2)


def _scatter_body(bpw, rows_per_tile, mt_ref, row_ref, zeros_ref, out_ref,
                  idx, mtv, acc, ssem):
    c = lax.axis_index("c")
    s = lax.axis_index("s")
    w = s * _NC + c
    t0 = s * rows_per_tile
    pltpu.sync_copy(zeros_ref.at[pl.ds(t0, rows_per_tile)],
                    acc.at[pl.ds(t0, rows_per_tile)])
    plsc.subcore_barrier()

    def chunk(q, carry):
        blk0 = w * bpw + q * _FIRE_S
        pltpu.sync_copy(row_ref.at[pl.ds(blk0, _FIRE_S)], idx)
        pltpu.sync_copy(mt_ref.at[pl.ds(blk0 * _IB, _FIRE_S * _IB)], mtv)
        descs = [
            pltpu.async_copy(mtv.at[pl.ds(j * _IB, _IB)],
                             acc.at[idx.at[j]], ssem, add=True)
            for j in range(_FIRE_S)
        ]
        for d in descs:
            d.wait()
        return carry

    lax.fori_loop(0, bpw // _FIRE_S, chunk, 0)
    plsc.subcore_barrier()
    pltpu.sync_copy(acc.at[pl.ds(t0, rows_per_tile)],
                    out_ref.at[c, pl.ds(t0, rows_per_tile)])


def _sc_scatter(mt, row2d, zeros_pad):
    nblk = row2d.shape[0]
    bpw = nblk // _NW
    n_pad = zeros_pad.shape[0]
    rows_per_tile = n_pad // _NS
    mesh = plsc.VectorSubcoreMesh(core_axis_name="c", subcore_axis_name="s")
    return pl.kernel(
        functools.partial(_scatter_body, bpw, rows_per_tile),
        out_type=jax.ShapeDtypeStruct((_NC, n_pad, _WIDE), jnp.float32),
        mesh=mesh,
        compiler_params=pltpu.CompilerParams(use_tc_tiling_on_sc=False),
        scratch_types=[
            pltpu.VMEM((_FIRE_S, _IB), jnp.int32),
            pltpu.VMEM((_FIRE_S * _IB, _WIDE), jnp.float32),
            pltpu.VMEM_SHARED((n_pad, _WIDE), jnp.float32),
            pltpu.SemaphoreType.DMA,
        ],
    )(mt, row2d, zeros_pad)


def _k5_body(nf_ref, coord_ref, a0_ref, a1_ref, a2_ref, a3_ref,
             wn1a_ref, wn1b_ref, bn1_ref,
             wn2_ref, bn2_ref, node_out_ref, coord_out_ref):
    nf = nf_ref[...]
    a = (a0_ref[...] + a1_ref[...]) + (a2_ref[...] + a3_ref[...])
    aggh = a[:, :_D]
    aggc = a[:, _D:_D + 3]
    cnt = a[:, _D + 3:_D + 4]
    h1 = _silu(jnp.dot(nf, wn1a_ref[...], preferred_element_type=jnp.float32)
               + jnp.dot(aggh, wn1b_ref[...],
                         preferred_element_type=jnp.float32)
               + bn1_ref[...])
    h2 = jnp.dot(h1, wn2_ref[...], preferred_element_type=jnp.float32) \
        + bn2_ref[...]
    node_out_ref[...] = nf + h2
    coord_out_ref[...] = coord_ref[...] + aggc / jnp.maximum(cnt, 1.0)


def _node_mlp(node_feat, coord, a0, a1, a2, a3, W_n1a, W_n1b, b_n1, W_n2,
              b_n2, bn=400):
    n = node_feat.shape[0]
    grid = n // bn
    return pl.pallas_call(
        _k5_body,
        grid=(grid,),
        in_specs=[
            pl.BlockSpec((bn, _D), lambda i: (i, 0)),
            pl.BlockSpec((bn, 3), lambda i: (i, 0)),
            pl.BlockSpec((bn, _WIDE), lambda i: (i, 0)),
            pl.BlockSpec((bn, _WIDE), lambda i: (i, 0)),
            pl.BlockSpec((bn, _WIDE), lambda i: (i, 0)),
            pl.BlockSpec((bn, _WIDE), lambda i: (i, 0)),
            pl.BlockSpec((_D, _D), lambda i: (0, 0)),
            pl.BlockSpec((_D, _D), lambda i: (0, 0)),
            pl.BlockSpec((1, _D), lambda i: (0, 0)),
            pl.BlockSpec((_D, _D), lambda i: (0, 0)),
            pl.BlockSpec((1, _D), lambda i: (0, 0)),
        ],
        out_specs=[
            pl.BlockSpec((bn, _D), lambda i: (i, 0)),
            pl.BlockSpec((bn, 3), lambda i: (i, 0)),
        ],
        out_shape=[
            jax.ShapeDtypeStruct((n, _D), jnp.float32),
            jax.ShapeDtypeStruct((n, 3), jnp.float32),
        ],
    )(node_feat, coord, a0, a1, a2, a3, W_n1a, W_n1b, b_n1, W_n2, b_n2)


def kernel(coord, node_feat, edge_index, W_e1, b_e1, W_e2, b_e2, W_c1, b_c1,
           W_c2, W_n1, b_n1, W_n2, b_n2):
    n = coord.shape[0]
    e = edge_index.shape[1]

    blk_unit = _NW * _FIRE * _IB
    e_pad = ((e + blk_unit - 1) // blk_unit) * blk_unit
    n_pad = ((n + _NS - 1) // _NS + 15) // 16 * 16 * _NS

    row = edge_index[0]
    col = edge_index[1]
    pad = jnp.full((e_pad - e,), n, jnp.int32)
    row2d = jnp.concatenate([row, pad]).reshape(e_pad // _IB, _IB)
    col2d = jnp.concatenate([col, pad]).reshape(e_pad // _IB, _IB)

    W_e1a = W_e1[:_D]
    W_e1b = W_e1[_D:2 * _D]
    w_r = W_e1[2 * _D:2 * _D + 1]
    b_e1r = b_e1.reshape(1, _D)

    ta, tb = _make_tables(node_feat, coord, W_e1a, W_e1b, b_e1r)
    rpad = ((0, n_pad - n), (0, 0))
    ta = jnp.pad(ta, rpad)
    tb = jnp.pad(tb, rpad)
    lpad = ((0, 0), (0, 160 - _WIDE))
    ta2 = jnp.concatenate([ta, ta])
    tb2 = jnp.concatenate([tb, tb])
    ta2 = jnp.pad(ta2, lpad).astype(jnp.bfloat16)
    tb2 = jnp.pad(tb2, lpad).astype(jnp.bfloat16)
    nblk = e_pad // _IB
    h = nblk // 2
    bpw_h = h // _NW
    core_off = ((jnp.arange(h, dtype=jnp.int32)[:, None] // bpw_h) % _NC) \
        * n_pad

    zeros_pad = jnp.zeros((n_pad, _WIDE), jnp.float32)
    halves = []
    for lo, hi in ((0, h), (h, nblk)):
        sp = _sc_gather(ta2, tb2, row2d[lo:hi] + core_off,
                        col2d[lo:hi] + core_off)
        mt = _edge_mlp(sp, w_r, W_e2, b_e2.reshape(1, _D), W_c1,
                       b_c1.reshape(1, _D), W_c2)
        halves.append(_sc_scatter(mt, row2d[lo:hi], zeros_pad))
    ag, ag2 = halves

    node_out, coord_out = _node_mlp(
        node_feat, coord, ag[0, :n], ag[1, :n], ag2[0, :n], ag2[1, :n],
        W_n1[:_D], W_n1[_D:], b_n1.reshape(1, _D), W_n2,
        b_n2.reshape(1, _D))
    return node_out, coord_out


# speedup vs baseline: 1.0460x; 1.0460x over previous
import functools

import jax
import jax.numpy as jnp
from jax import lax
from jax.experimental import pallas as pl
from jax.experimental.pallas import tpu as pltpu
from jax.experimental.pallas import tpu_sc as plsc

_NC = 2
_NS = 16
_NW = _NC * _NS

_D = 128
_WIDE = 144
_IB = 128
_FIRE = 4
_FIRE_S = 2


def _silu(x):
    return x / (1.0 + jnp.exp(-x))


def _k1_body(nf_ref, coord_ref, wa_ref, wb_ref, be1_ref, ta_ref, tb_ref):
    nf = nf_ref[...]
    cd = coord_ref[...]
    bn = nf.shape[0]
    zpad = jnp.zeros((bn, _WIDE - _D - 3), jnp.float32)
    a = jnp.dot(nf, wa_ref[...], preferred_element_type=jnp.float32) + be1_ref[...]
    b = jnp.dot(nf, wb_ref[...], preferred_element_type=jnp.float32)
    ta_ref[...] = jnp.concatenate([a, cd, zpad], axis=1)
    tb_ref[...] = jnp.concatenate([b, -cd, zpad], axis=1)


def _make_tables(node_feat, coord, W_e1a, W_e1b, b_e1, bn=400):
    n = node_feat.shape[0]
    grid = n // bn
    return pl.pallas_call(
        _k1_body,
        grid=(grid,),
        in_specs=[
            pl.BlockSpec((bn, _D), lambda i: (i, 0)),
            pl.BlockSpec((bn, 3), lambda i: (i, 0)),
            pl.BlockSpec((_D, _D), lambda i: (0, 0)),
            pl.BlockSpec((_D, _D), lambda i: (0, 0)),
            pl.BlockSpec((1, _D), lambda i: (0, 0)),
        ],
        out_specs=[
            pl.BlockSpec((bn, _WIDE), lambda i: (i, 0)),
            pl.BlockSpec((bn, _WIDE), lambda i: (i, 0)),
        ],
        out_shape=[
            jax.ShapeDtypeStruct((n, _WIDE), jnp.float32),
            jax.ShapeDtypeStruct((n, _WIDE), jnp.float32),
        ],
    )(node_feat, coord, W_e1a, W_e1b, b_e1)


def _gather_body(bpw, ta_ref, tb_ref, row_ref, col_ref, out_ref,
                 rbuf, cbuf, sbuf, gsem):
    w = lax.axis_index("s") * _NC + lax.axis_index("c")
    base = w * bpw
    pltpu.sync_copy(row_ref.at[pl.ds(base, bpw)], rbuf)
    pltpu.sync_copy(col_ref.at[pl.ds(base, bpw)], cbuf)

    def chunk(q, carry):
        blk0 = q * _FIRE
        descs = [
            pltpu.async_copy(ta_ref.at[rbuf.at[blk0 + j]],
                             sbuf.at[pl.ds(j * _IB, _IB)], gsem)
            for j in range(_FIRE)
        ]
        for d in descs:
            d.wait()
        descs = [
            pltpu.async_copy(tb_ref.at[cbuf.at[blk0 + j]],
                             sbuf.at[pl.ds(j * _IB, _IB)], gsem, add=True)
            for j in range(_FIRE)
        ]
        for d in descs:
            d.wait()
        pltpu.sync_copy(sbuf,
                        out_ref.at[pl.ds((base + blk0) * _IB, _FIRE * _IB)])
        return carry

    lax.fori_loop(0, bpw // _FIRE, chunk, 0)


def _sc_gather(ta, tb, row2d, col2d):
    nblk = row2d.shape[0]
    bpw = nblk // _NW
    e_pad = nblk * _IB
    wid = ta.shape[1]
    mesh = plsc.VectorSubcoreMesh(core_axis_name="c", subcore_axis_name="s")
    return pl.kernel(
        functools.partial(_gather_body, bpw),
        out_type=jax.ShapeDtypeStruct((e_pad, wid), ta.dtype),
        mesh=mesh,
        compiler_params=pltpu.CompilerParams(use_tc_tiling_on_sc=False),
        scratch_types=[
            pltpu.VMEM((bpw, _IB), jnp.int32),
            pltpu.VMEM((bpw, _IB), jnp.int32),
            pltpu.VMEM((_FIRE * _IB, wid), ta.dtype),
            pltpu.SemaphoreType.DMA,
        ],
    )(ta, tb, row2d, col2d)


def _k2_body(sp_ref, wr_ref, we2_ref, be2_ref, wc1_ref, bc1_ref, wc2_ref,
             out_ref):
    sp = sp_ref[...].astype(jnp.float32)
    bn = sp.shape[0]
    x = sp[:, :_D]
    cd = sp[:, _D:_D + 3]
    r = jnp.sum(cd * cd, axis=1, keepdims=True)
    pre = x + r * wr_ref[...]
    h1 = _silu(pre)
    m = _silu(jnp.dot(h1, we2_ref[...], preferred_element_type=jnp.float32)
              + be2_ref[...])
    g1 = _silu(jnp.dot(m, wc1_ref[...], preferred_element_type=jnp.float32)
               + bc1_ref[...])
    gate = jnp.dot(g1, wc2_ref[...], preferred_element_type=jnp.float32)
    trans = cd * gate
    ones = jnp.ones((bn, 1), jnp.float32)
    zpad = jnp.zeros((bn, _WIDE - _D - 4), jnp.float32)
    out_ref[...] = jnp.concatenate([m, trans, ones, zpad], axis=1)


def _edge_mlp(sp, w_r, W_e2, b_e2, W_c1, b_c1, W_c2, bn=2048):
    e_pad = sp.shape[0]
    grid = e_pad // bn
    wid = sp.shape[1]
    return pl.pallas_call(
        _k2_body,
        grid=(grid,),
        in_specs=[
            pl.BlockSpec((bn, wid), lambda i: (i, 0)),
            pl.BlockSpec((1, _D), lambda i: (0, 0)),
            pl.BlockSpec((_D, _D), lambda i: (0, 0)),
            pl.BlockSpec((1, _D), lambda i: (0, 0)),
            pl.BlockSpec((_D, _D), lambda i: (0, 0)),
            pl.BlockSpec((1, _D), lambda i: (0, 0)),
            pl.BlockSpec((_D, 1), lambda i: (0, 0)),
        ],
        out_specs=pl.BlockSpec((bn, _WIDE), lambda i: (i, 0)),
        out_shape=jax.ShapeDtypeStruct((e_pad, _WIDE), jnp.float32),
    )(sp, w_r, W_e2, b_e2, W_c1, b_c1, W_c2)


def _scatter_body(bpw, rows_per_tile, mt_ref, row_ref, zeros_ref, out_ref,
                  idx, mtv, acc, ssem):
    c = lax.axis_index("c")
    s = lax.axis_index("s")
    w = s * _NC + c
    t0 = s * rows_per_tile
    pltpu.sync_copy(zeros_ref.at[pl.ds(t0, rows_per_tile)],
                    acc.at[pl.ds(t0, rows_per_tile)])
    plsc.subcore_barrier()

    def chunk(q, carry):
        blk0 = w * bpw + q * _FIRE_S
        pltpu.sync_copy(row_ref.at[pl.ds(blk0, _FIRE_S)], idx)
        pltpu.sync_copy(mt_ref.at[pl.ds(blk0 * _IB, _FIRE_S * _IB)], mtv)
        descs = [
            pltpu.async_copy(mtv.at[pl.ds(j * _IB, _IB)],
                             acc.at[idx.at[j]], ssem, add=True)
            for j in range(_FIRE_S)
        ]
        for d in descs:
            d.wait()
        return carry

    lax.fori_loop(0, bpw // _FIRE_S, chunk, 0)
    plsc.subcore_barrier()
    pltpu.sync_copy(acc.at[pl.ds(t0, rows_per_tile)],
                    out_ref.at[c, pl.ds(t0, rows_per_tile)])


def _sc_scatter(mt, row2d, zeros_pad):
    nblk = row2d.shape[0]
    bpw = nblk // _NW
    n_pad = zeros_pad.shape[0]
    rows_per_tile = n_pad // _NS
    mesh = plsc.VectorSubcoreMesh(core_axis_name="c", subcore_axis_name="s")
    return pl.kernel(
        functools.partial(_scatter_body, bpw, rows_per_tile),
        out_type=jax.ShapeDtypeStruct((_NC, n_pad, _WIDE), jnp.float32),
        mesh=mesh,
        compiler_params=pltpu.CompilerParams(use_tc_tiling_on_sc=False),
        scratch_types=[
            pltpu.VMEM((_FIRE_S, _IB), jnp.int32),
            pltpu.VMEM((_FIRE_S * _IB, _WIDE), jnp.float32),
            pltpu.VMEM_SHARED((n_pad, _WIDE), jnp.float32),
            pltpu.SemaphoreType.DMA,
        ],
    )(mt, row2d, zeros_pad)


def _k5_body(nf_ref, coord_ref, a0_ref, a1_ref, a2_ref, a3_ref,
             wn1a_ref, wn1b_ref, bn1_ref,
             wn2_ref, bn2_ref, node_out_ref, coord_out_ref):
    nf = nf_ref[...]
    a = (a0_ref[...] + a1_ref[...]) + (a2_ref[...] + a3_ref[...])
    aggh = a[:, :_D]
    aggc = a[:, _D:_D + 3]
    cnt = a[:, _D + 3:_D + 4]
    h1 = _silu(jnp.dot(nf, wn1a_ref[...], preferred_element_type=jnp.float32)
               + jnp.dot(aggh, wn1b_ref[...],
                         preferred_element_type=jnp.float32)
               + bn1_ref[...])
    h2 = jnp.dot(h1, wn2_ref[...], preferred_element_type=jnp.float32) \
        + bn2_ref[...]
    node_out_ref[...] = nf + h2
    coord_out_ref[...] = coord_ref[...] + aggc / jnp.maximum(cnt, 1.0)


def _node_mlp(node_feat, coord, a0, a1, a2, a3, W_n1a, W_n1b, b_n1, W_n2,
              b_n2, bn=400):
    n = node_feat.shape[0]
    grid = n // bn
    return pl.pallas_call(
        _k5_body,
        grid=(grid,),
        in_specs=[
            pl.BlockSpec((bn, _D), lambda i: (i, 0)),
            pl.BlockSpec((bn, 3), lambda i: (i, 0)),
            pl.BlockSpec((bn, _WIDE), lambda i: (i, 0)),
            pl.BlockSpec((bn, _WIDE), lambda i: (i, 0)),
            pl.BlockSpec((bn, _WIDE), lambda i: (i, 0)),
            pl.BlockSpec((bn, _WIDE), lambda i: (i, 0)),
            pl.BlockSpec((_D, _D), lambda i: (0, 0)),
            pl.BlockSpec((_D, _D), lambda i: (0, 0)),
            pl.BlockSpec((1, _D), lambda i: (0, 0)),
            pl.BlockSpec((_D, _D), lambda i: (0, 0)),
            pl.BlockSpec((1, _D), lambda i: (0, 0)),
        ],
        out_specs=[
            pl.BlockSpec((bn, _D), lambda i: (i, 0)),
            pl.BlockSpec((bn, 3), lambda i: (i, 0)),
        ],
        out_shape=[
            jax.ShapeDtypeStruct((n, _D), jnp.float32),
            jax.ShapeDtypeStruct((n, 3), jnp.float32),
        ],
    )(node_feat, coord, a0, a1, a2, a3, W_n1a, W_n1b, b_n1, W_n2, b_n2)


def kernel(coord, node_feat, edge_index, W_e1, b_e1, W_e2, b_e2, W_c1, b_c1,
           W_c2, W_n1, b_n1, W_n2, b_n2):
    n = coord.shape[0]
    e = edge_index.shape[1]

    blk_unit = _NW * _FIRE * _IB
    e_pad = ((e + blk_unit - 1) // blk_unit) * blk_unit
    n_pad = ((n + _NS - 1) // _NS + 15) // 16 * 16 * _NS

    row = edge_index[0]
    col = edge_index[1]
    pad = jnp.full((e_pad - e,), n, jnp.int32)
    row2d = jnp.concatenate([row, pad]).reshape(e_pad // _IB, _IB)
    col2d = jnp.concatenate([col, pad]).reshape(e_pad // _IB, _IB)

    W_e1a = W_e1[:_D]
    W_e1b = W_e1[_D:2 * _D]
    w_r = W_e1[2 * _D:2 * _D + 1]
    b_e1r = b_e1.reshape(1, _D)

    ta, tb = _make_tables(node_feat, coord, W_e1a, W_e1b, b_e1r)
    rpad = ((0, n_pad - n), (0, 0))
    ta = jnp.pad(ta, rpad)
    tb = jnp.pad(tb, rpad)
    ta2 = jnp.concatenate([ta, ta])
    tb2 = jnp.concatenate([tb, tb])
    nblk = e_pad // _IB
    h = nblk // 2
    bpw_h = h // _NW
    core_off = ((jnp.arange(h, dtype=jnp.int32)[:, None] // bpw_h) % _NC) \
        * n_pad

    zeros_pad = jnp.zeros((n_pad, _WIDE), jnp.float32)
    halves = []
    for lo, hi in ((0, h), (h, nblk)):
        sp = _sc_gather(ta2, tb2, row2d[lo:hi] + core_off,
                        col2d[lo:hi] + core_off)
        mt = _edge_mlp(sp, w_r, W_e2, b_e2.reshape(1, _D), W_c1,
                       b_c1.reshape(1, _D), W_c2)
        halves.append(_sc_scatter(mt, row2d[lo:hi], zeros_pad))
    ag, ag2 = halves

    node_out, coord_out = _node_mlp(
        node_feat, coord, ag[0, :n], ag[1, :n], ag2[0, :n], ag2[1, :n],
        W_n1[:_D], W_n1[_D:], b_n1.reshape(1, _D), W_n2,
        b_n2.reshape(1, _D))
    return node_out, coord_out
